# baseline (speedup 1.0000x reference)
import functools

import jax
import jax.numpy as jnp
from jax import lax
from jax.experimental import pallas as pl
from jax.experimental.pallas import tpu as pltpu
from jax.experimental.pallas import tpu_sc as plsc

B, S, V, P, D = 1024, 200, 50265, 514, 768
PAD_IDX = 1
N = B * S
LANES = 16
DV = D // LANES
KB = 32


def _kernel_body(ids_hbm, tt_hbm, tok_hbm, pos_hbm, ent_hbm, gam_hbm, bet_hbm,
                 out_hbm,
                 idx_t, idx_p, idx_e, tok_buf, pos_buf, ent_buf,
                 gam_buf, bet_buf, sem_t, sem_p, sem_e):
    nc = 2
    wid = lax.axis_index("s") * nc + lax.axis_index("c")
    per_w = N // 32
    nblk = per_w // KB

    pltpu.sync_copy(gam_hbm, gam_buf)
    pltpu.sync_copy(bet_hbm, bet_buf)

    def block_body(g, _):
        base = wid * per_w + g * KB

        pltpu.sync_copy(ids_hbm.at[pl.ds(base, KB)], idx_t)
        pltpu.sync_copy(tt_hbm.at[pl.ds(base, KB)], idx_e)
        for j in range(KB // LANES):
            lane = lax.iota(jnp.int32, LANES)
            pvec = lax.rem(base + j * LANES + lane, S) + (PAD_IDX + 1)
            idx_p[pl.ds(j * LANES, LANES)] = pvec

        ct = pltpu.async_copy(tok_hbm.at[idx_t], tok_buf, sem_t)
        cp = pltpu.async_copy(pos_hbm.at[idx_p], pos_buf, sem_p)
        ce = pltpu.async_copy(ent_hbm.at[idx_e], ent_buf, sem_e)
        ct.wait()
        cp.wait()
        ce.wait()

        def token_body(i, _):
            def acc_body(j, carry):
                acc_s, acc_q = carry
                off = j * LANES
                v = (tok_buf[i, pl.ds(off, LANES)]
                     + pos_buf[i, pl.ds(off, LANES)]
                     + ent_buf[i, pl.ds(off, LANES)])
                tok_buf[i, pl.ds(off, LANES)] = v
                return acc_s + v, acc_q + v * v

            zeros = jnp.zeros((LANES,), jnp.float32)
            acc_s, acc_q = lax.fori_loop(0, DV, acc_body, (zeros, zeros))
            lane = lax.iota(jnp.int32, LANES)
            dnums = lax.GatherDimensionNumbers(
                offset_dims=(), collapsed_slice_dims=(0,),
                start_index_map=(0,))
            for sh in (8, 4, 2, 1):
                perm = lax.bitwise_xor(lane, jnp.int32(sh))[:, None]
                acc_s = acc_s + lax.gather(
                    acc_s, perm, dnums, slice_sizes=(1,),
                    mode=lax.GatherScatterMode.PROMISE_IN_BOUNDS)
                acc_q = acc_q + lax.gather(
                    acc_q, perm, dnums, slice_sizes=(1,),
                    mode=lax.GatherScatterMode.PROMISE_IN_BOUNDS)
            mean_v = acc_s * (1.0 / D)
            xv = acc_q * (1.0 / D) - mean_v * mean_v + 1e-5
            bits = lax.bitcast_convert_type(xv, jnp.int32)
            y = lax.bitcast_convert_type(
                jnp.int32(0x5F3759DF) - lax.shift_right_arithmetic(
                    bits, jnp.int32(1)), jnp.float32)
            hx = xv * 0.5
            for _ in range(3):
                y = y * (1.5 - hx * y * y)

            def norm_body(j, _):
                off = j * LANES
                v = tok_buf[i, pl.ds(off, LANES)]
                o = ((v - mean_v) * y * gam_buf[pl.ds(off, LANES)]
                     + bet_buf[pl.ds(off, LANES)])
                tok_buf[i, pl.ds(off, LANES)] = o
                return 0

            lax.fori_loop(0, DV, norm_body, 0)
            return 0

        lax.fori_loop(0, KB, token_body, 0)

        pltpu.sync_copy(tok_buf, out_hbm.at[pl.ds(base, KB)])
        return 0

    lax.fori_loop(0, nblk, block_body, 0)


@jax.jit
def _run(ids32, tt32, tok_table, pos_table, ent_table, ln_gamma, ln_beta):
    mesh = plsc.VectorSubcoreMesh(core_axis_name="c", subcore_axis_name="s")
    f = pl.kernel(
        _kernel_body,
        out_type=jax.ShapeDtypeStruct((N, D), jnp.float32),
        mesh=mesh,
        scratch_types=[
            pltpu.VMEM((KB,), jnp.int32),
            pltpu.VMEM((KB,), jnp.int32),
            pltpu.VMEM((KB,), jnp.int32),
            pltpu.VMEM((KB, D), jnp.float32),
            pltpu.VMEM((KB, D), jnp.float32),
            pltpu.VMEM((KB, D), jnp.float32),
            pltpu.VMEM((D,), jnp.float32),
            pltpu.VMEM((D,), jnp.float32),
            pltpu.SemaphoreType.DMA,
            pltpu.SemaphoreType.DMA,
            pltpu.SemaphoreType.DMA,
        ],
    )
    return f(ids32, tt32, tok_table, pos_table, ent_table, ln_gamma, ln_beta)


def kernel(input_ids, token_type_ids, tok_table, pos_table, ent_table,
           ln_gamma, ln_beta):
    ids32 = input_ids.reshape(-1).astype(jnp.int32)
    tt32 = token_type_ids.reshape(-1).astype(jnp.int32)
    out = _run(ids32, tt32, tok_table, pos_table, ent_table, ln_gamma, ln_beta)
    return out.reshape(B, S, D)


# speedup vs baseline: 7.3268x; 7.3268x over previous
import jax
import jax.numpy as jnp
from jax import lax
from jax.experimental import pallas as pl
from jax.experimental.pallas import tpu as pltpu
from jax.experimental.pallas import tpu_sc as plsc

B, S, V, P, D = 1024, 200, 50265, 514, 768
PAD_IDX = 1
N = B * S
NW = 32
KB = 64
PER_W = N // NW
NBLK = PER_W // KB


def _sc_gather_body(ids_hbm, tok_hbm, out_hbm,
                    idx0, idx1, buf0, buf1,
                    gsem0, gsem1, wsem0, wsem1):
    nc = 2
    wid = lax.axis_index("s") * nc + lax.axis_index("c")
    wbase = wid * PER_W

    idx = (idx0, idx1)
    buf = (buf0, buf1)
    gsem = (gsem0, gsem1)
    wsem = (wsem0, wsem1)

    pltpu.sync_copy(ids_hbm.at[pl.ds(wbase, KB)], idx0)
    pltpu.async_copy(tok_hbm.at[idx0], buf0, gsem0)

    def pair_body(h, _):
        for sub in (0, 1):
            g = 2 * h + sub
            cur, nxt = sub, 1 - sub

            def wait_prev_write():
                pltpu.make_async_copy(
                    buf[nxt], out_hbm.at[pl.ds(0, KB)], wsem[nxt]).wait()

            if sub == 1:
                wait_prev_write()
            else:
                pl.when(h > 0)(wait_prev_write)

            def launch_next():
                nbase = wbase + (g + 1) * KB
                pltpu.sync_copy(ids_hbm.at[pl.ds(nbase, KB)], idx[nxt])
                pltpu.async_copy(tok_hbm.at[idx[nxt]], buf[nxt], gsem[nxt])

            if sub == 0:
                launch_next()
            else:
                pl.when(h < NBLK // 2 - 1)(launch_next)

            pltpu.make_async_copy(
                tok_hbm.at[idx[cur]], buf[cur], gsem[cur]).wait()
            pltpu.async_copy(
                buf[cur], out_hbm.at[pl.ds(wbase + g * KB, KB)], wsem[cur])
        return 0

    lax.fori_loop(0, NBLK // 2, pair_body, 0)
    pltpu.make_async_copy(
        buf1, out_hbm.at[pl.ds(0, KB)], wsem1).wait()


def _tc_ln_body(g_ref, tt_ref, pos_ref, ent_ref, gam_ref, bet_ref, o_ref):
    x = g_ref[...] + pos_ref[...][None]
    ttf = tt_ref[...].astype(jnp.float32)[..., None]
    e0 = ent_ref[0, :][None, None, :]
    de = (ent_ref[1, :] - ent_ref[0, :])[None, None, :]
    x = x + e0 + ttf * de
    mean = jnp.mean(x, axis=-1, keepdims=True)
    xc = x - mean
    var = jnp.mean(xc * xc, axis=-1, keepdims=True)
    o_ref[...] = (xc * lax.rsqrt(var + 1e-5) * gam_ref[...][None, None, :]
                  + bet_ref[...][None, None, :])


@jax.jit
def _run(ids32, tt32, tok_table, pos_table, ent_table, ln_gamma, ln_beta):
    mesh = plsc.VectorSubcoreMesh(core_axis_name="c", subcore_axis_name="s")
    gather = pl.kernel(
        _sc_gather_body,
        out_type=jax.ShapeDtypeStruct((N, D), jnp.float32),
        mesh=mesh,
        scratch_types=[
            pltpu.VMEM((KB,), jnp.int32),
            pltpu.VMEM((KB,), jnp.int32),
            pltpu.VMEM((KB, D), jnp.float32),
            pltpu.VMEM((KB, D), jnp.float32),
            pltpu.SemaphoreType.DMA,
            pltpu.SemaphoreType.DMA,
            pltpu.SemaphoreType.DMA,
            pltpu.SemaphoreType.DMA,
        ],
    )
    tok_rows = gather(ids32, tok_table)

    tok_rows = tok_rows.reshape(B, S, D)
    pos_slice = lax.slice_in_dim(pos_table, PAD_IDX + 1, PAD_IDX + 1 + S,
                                 axis=0)
    bs = 8
    out = pl.pallas_call(
        _tc_ln_body,
        grid=(B // bs,),
        in_specs=[
            pl.BlockSpec((bs, S, D), lambda i: (i, 0, 0)),
            pl.BlockSpec((bs, S), lambda i: (i, 0)),
            pl.BlockSpec((S, D), lambda i: (0, 0)),
            pl.BlockSpec((2, D), lambda i: (0, 0)),
            pl.BlockSpec((D,), lambda i: (0,)),
            pl.BlockSpec((D,), lambda i: (0,)),
        ],
        out_specs=pl.BlockSpec((bs, S, D), lambda i: (i, 0, 0)),
        out_shape=jax.ShapeDtypeStruct((B, S, D), jnp.float32),
    )(tok_rows, tt32.reshape(B, S), pos_slice, ent_table, ln_gamma, ln_beta)
    return out


def kernel(input_ids, token_type_ids, tok_table, pos_table, ent_table,
           ln_gamma, ln_beta):
    ids32 = input_ids.reshape(-1).astype(jnp.int32)
    tt32 = token_type_ids.reshape(-1).astype(jnp.int32)
    return _run(ids32, tt32, tok_table, pos_table, ent_table,
                ln_gamma, ln_beta)


# speedup vs baseline: 7.3961x; 1.0095x over previous
import jax
import jax.numpy as jnp
from jax import lax
from jax.experimental import pallas as pl
from jax.experimental.pallas import tpu as pltpu
from jax.experimental.pallas import tpu_sc as plsc

B, S, V, P, D = 1024, 200, 50265, 514, 768
PAD_IDX = 1
N = B * S
NW = 32
NCHUNK = 4
BC = B // NCHUNK
NC_TOK = BC * S
KB = 40
PER_W = NC_TOK // NW
NBLK = PER_W // KB
BS_TC = 8


def _sc_gather_body(ids_hbm, tok_hbm, out_hbm,
                    idx0, idx1, buf0, buf1,
                    gsem0, gsem1, wsem0, wsem1):
    nc = 2
    wid = lax.axis_index("s") * nc + lax.axis_index("c")
    wbase = wid * PER_W

    idx = (idx0, idx1)
    buf = (buf0, buf1)
    gsem = (gsem0, gsem1)
    wsem = (wsem0, wsem1)

    pltpu.sync_copy(ids_hbm.at[pl.ds(wbase, KB)], idx0)
    pltpu.async_copy(tok_hbm.at[idx0], buf0, gsem0)

    def pair_body(h, _):
        for sub in (0, 1):
            g = 2 * h + sub
            cur, nxt = sub, 1 - sub

            def wait_prev_write():
                pltpu.make_async_copy(
                    buf[nxt], out_hbm.at[pl.ds(0, KB)], wsem[nxt]).wait()

            if sub == 1:
                wait_prev_write()
            else:
                pl.when(h > 0)(wait_prev_write)

            def launch_next():
                nbase = wbase + (g + 1) * KB
                pltpu.sync_copy(ids_hbm.at[pl.ds(nbase, KB)], idx[nxt])
                pltpu.async_copy(tok_hbm.at[idx[nxt]], buf[nxt], gsem[nxt])

            if sub == 0:
                launch_next()
            else:
                pl.when(h < NBLK // 2 - 1)(launch_next)

            pltpu.make_async_copy(
                tok_hbm.at[idx[cur]], buf[cur], gsem[cur]).wait()
            pltpu.async_copy(
                buf[cur], out_hbm.at[pl.ds(wbase + g * KB, KB)], wsem[cur])
        return 0

    lax.fori_loop(0, NBLK // 2, pair_body, 0)
    pltpu.make_async_copy(
        buf1, out_hbm.at[pl.ds(0, KB)], wsem1).wait()


def _tc_ln_body(g_ref, tt_ref, pos_ref, ent_ref, gam_ref, bet_ref, o_ref):
    x = g_ref[...] + pos_ref[...][None]
    ttf = tt_ref[...].astype(jnp.float32)[..., None]
    e0 = ent_ref[0, :][None, None, :]
    de = (ent_ref[1, :] - ent_ref[0, :])[None, None, :]
    x = x + e0 + ttf * de
    mean = jnp.mean(x, axis=-1, keepdims=True)
    xc = x - mean
    var = jnp.mean(xc * xc, axis=-1, keepdims=True)
    o_ref[...] = (xc * lax.rsqrt(var + 1e-5) * gam_ref[...][None, None, :]
                  + bet_ref[...][None, None, :])


def _tc_ln_body_aliased(g_ref, tt_ref, pos_ref, ent_ref, gam_ref, bet_ref,
                        buf_ref, o_ref):
    del buf_ref
    _tc_ln_body(g_ref, tt_ref, pos_ref, ent_ref, gam_ref, bet_ref, o_ref)


@jax.jit
def _run(ids32, tt32, tok_table, pos_table, ent_table, ln_gamma, ln_beta):
    mesh = plsc.VectorSubcoreMesh(core_axis_name="c", subcore_axis_name="s")
    gather = pl.kernel(
        _sc_gather_body,
        out_type=jax.ShapeDtypeStruct((NC_TOK, D), jnp.float32),
        mesh=mesh,
        scratch_types=[
            pltpu.VMEM((KB,), jnp.int32),
            pltpu.VMEM((KB,), jnp.int32),
            pltpu.VMEM((KB, D), jnp.float32),
            pltpu.VMEM((KB, D), jnp.float32),
            pltpu.SemaphoreType.DMA,
            pltpu.SemaphoreType.DMA,
            pltpu.SemaphoreType.DMA,
            pltpu.SemaphoreType.DMA,
        ],
    )

    pos_slice = lax.slice_in_dim(pos_table, PAD_IDX + 1, PAD_IDX + 1 + S,
                                 axis=0)
    tt_bs = tt32.reshape(B, S)
    grid_c = BC // BS_TC

    def tc_chunk(c, rows_c, buf):
        common_in_specs = [
            pl.BlockSpec((BS_TC, S, D), lambda i: (i, 0, 0)),
            pl.BlockSpec((BS_TC, S), lambda i, c=c: (c * grid_c + i, 0)),
            pl.BlockSpec((S, D), lambda i: (0, 0)),
            pl.BlockSpec((2, D), lambda i: (0, 0)),
            pl.BlockSpec((D,), lambda i: (0,)),
            pl.BlockSpec((D,), lambda i: (0,)),
        ]
        out_spec = pl.BlockSpec((BS_TC, S, D),
                                lambda i, c=c: (c * grid_c + i, 0, 0))
        args = (rows_c.reshape(BC, S, D), tt_bs, pos_slice, ent_table,
                ln_gamma, ln_beta)
        if buf is None:
            return pl.pallas_call(
                _tc_ln_body,
                grid=(grid_c,),
                in_specs=common_in_specs,
                out_specs=out_spec,
                out_shape=jax.ShapeDtypeStruct((B, S, D), jnp.float32),
            )(*args)
        return pl.pallas_call(
            _tc_ln_body_aliased,
            grid=(grid_c,),
            in_specs=common_in_specs + [pl.BlockSpec(memory_space=pl.ANY)],
            out_specs=out_spec,
            out_shape=jax.ShapeDtypeStruct((B, S, D), jnp.float32),
            input_output_aliases={6: 0},
        )(*args, buf)

    out = None
    for c in range(NCHUNK):
        ids_c = lax.slice_in_dim(ids32, c * NC_TOK, (c + 1) * NC_TOK, axis=0)
        rows_c = gather(ids_c, tok_table)
        out = tc_chunk(c, rows_c, out)
    return out


def kernel(input_ids, token_type_ids, tok_table, pos_table, ent_table,
           ln_gamma, ln_beta):
    ids32 = input_ids.reshape(-1).astype(jnp.int32)
    tt32 = token_type_ids.reshape(-1).astype(jnp.int32)
    return _run(ids32, tt32, tok_table, pos_table, ent_table,
                ln_gamma, ln_beta)


# speedup vs baseline: 7.3995x; 1.0005x over previous
import jax
import jax.numpy as jnp
from jax import lax
from jax.experimental import pallas as pl
from jax.experimental.pallas import tpu as pltpu
from jax.experimental.pallas import tpu_sc as plsc

B, S, V, P, D = 1024, 200, 50265, 514, 768
PAD_IDX = 1
N = B * S
NW = 32
NCHUNK = 4
BC = B // NCHUNK
NC_TOK = BC * S
KB = 80
PER_W = NC_TOK // NW
NBLK = PER_W // KB
BS_TC = 8


def _sc_gather_body(ids_hbm, tok_hbm, out_hbm,
                    idx0, idx1, buf0, buf1,
                    gsem0, gsem1, wsem0, wsem1):
    nc = 2
    wid = lax.axis_index("s") * nc + lax.axis_index("c")
    wbase = wid * PER_W

    idx = (idx0, idx1)
    buf = (buf0, buf1)
    gsem = (gsem0, gsem1)
    wsem = (wsem0, wsem1)

    pltpu.sync_copy(ids_hbm.at[pl.ds(wbase, KB)], idx0)
    pltpu.async_copy(tok_hbm.at[idx0], buf0, gsem0)

    def pair_body(h, _):
        for sub in (0, 1):
            g = 2 * h + sub
            cur, nxt = sub, 1 - sub

            def wait_prev_write():
                pltpu.make_async_copy(
                    buf[nxt], out_hbm.at[pl.ds(0, KB)], wsem[nxt]).wait()

            if sub == 1:
                wait_prev_write()
            else:
                pl.when(h > 0)(wait_prev_write)

            def launch_next():
                nbase = wbase + (g + 1) * KB
                pltpu.sync_copy(ids_hbm.at[pl.ds(nbase, KB)], idx[nxt])
                pltpu.async_copy(tok_hbm.at[idx[nxt]], buf[nxt], gsem[nxt])

            if sub == 0:
                launch_next()
            else:
                pl.when(h < NBLK // 2 - 1)(launch_next)

            pltpu.make_async_copy(
                tok_hbm.at[idx[cur]], buf[cur], gsem[cur]).wait()
            pltpu.async_copy(
                buf[cur], out_hbm.at[pl.ds(wbase + g * KB, KB)], wsem[cur])
        return 0

    lax.fori_loop(0, NBLK // 2, pair_body, 0)
    pltpu.make_async_copy(
        buf1, out_hbm.at[pl.ds(0, KB)], wsem1).wait()


def _tc_ln_body(g_ref, tt_ref, pos_ref, ent_ref, gam_ref, bet_ref, o_ref):
    x = g_ref[...] + pos_ref[...][None]
    ttf = tt_ref[...].astype(jnp.float32)[..., None]
    e0 = ent_ref[0, :][None, None, :]
    de = (ent_ref[1, :] - ent_ref[0, :])[None, None, :]
    x = x + e0 + ttf * de
    mean = jnp.mean(x, axis=-1, keepdims=True)
    xc = x - mean
    var = jnp.mean(xc * xc, axis=-1, keepdims=True)
    o_ref[...] = (xc * lax.rsqrt(var + 1e-5) * gam_ref[...][None, None, :]
                  + bet_ref[...][None, None, :])


def _tc_ln_body_aliased(g_ref, tt_ref, pos_ref, ent_ref, gam_ref, bet_ref,
                        buf_ref, o_ref):
    del buf_ref
    _tc_ln_body(g_ref, tt_ref, pos_ref, ent_ref, gam_ref, bet_ref, o_ref)


@jax.jit
def _run(ids32, tt32, tok_table, pos_table, ent_table, ln_gamma, ln_beta):
    mesh = plsc.VectorSubcoreMesh(core_axis_name="c", subcore_axis_name="s")
    gather = pl.kernel(
        _sc_gather_body,
        out_type=jax.ShapeDtypeStruct((NC_TOK, D), jnp.float32),
        mesh=mesh,
        scratch_types=[
            pltpu.VMEM((KB,), jnp.int32),
            pltpu.VMEM((KB,), jnp.int32),
            pltpu.VMEM((KB, D), jnp.float32),
            pltpu.VMEM((KB, D), jnp.float32),
            pltpu.SemaphoreType.DMA,
            pltpu.SemaphoreType.DMA,
            pltpu.SemaphoreType.DMA,
            pltpu.SemaphoreType.DMA,
        ],
    )

    pos_slice = lax.slice_in_dim(pos_table, PAD_IDX + 1, PAD_IDX + 1 + S,
                                 axis=0)
    tt_bs = tt32.reshape(B, S)
    grid_c = BC // BS_TC

    def tc_chunk(c, rows_c, buf):
        common_in_specs = [
            pl.BlockSpec((BS_TC, S, D), lambda i: (i, 0, 0)),
            pl.BlockSpec((BS_TC, S), lambda i, c=c: (c * grid_c + i, 0)),
            pl.BlockSpec((S, D), lambda i: (0, 0)),
            pl.BlockSpec((2, D), lambda i: (0, 0)),
            pl.BlockSpec((D,), lambda i: (0,)),
            pl.BlockSpec((D,), lambda i: (0,)),
        ]
        out_spec = pl.BlockSpec((BS_TC, S, D),
                                lambda i, c=c: (c * grid_c + i, 0, 0))
        args = (rows_c.reshape(BC, S, D), tt_bs, pos_slice, ent_table,
                ln_gamma, ln_beta)
        if buf is None:
            return pl.pallas_call(
                _tc_ln_body,
                grid=(grid_c,),
                in_specs=common_in_specs,
                out_specs=out_spec,
                out_shape=jax.ShapeDtypeStruct((B, S, D), jnp.float32),
            )(*args)
        return pl.pallas_call(
            _tc_ln_body_aliased,
            grid=(grid_c,),
            in_specs=common_in_specs + [pl.BlockSpec(memory_space=pl.ANY)],
            out_specs=out_spec,
            out_shape=jax.ShapeDtypeStruct((B, S, D), jnp.float32),
            input_output_aliases={6: 0},
        )(*args, buf)

    out = None
    for c in range(NCHUNK):
        ids_c = lax.slice_in_dim(ids32, c * NC_TOK, (c + 1) * NC_TOK, axis=0)
        rows_c = gather(ids_c, tok_table)
        out = tc_chunk(c, rows_c, out)
    return out


def kernel(input_ids, token_type_ids, tok_table, pos_table, ent_table,
           ln_gamma, ln_beta):
    ids32 = input_ids.reshape(-1).astype(jnp.int32)
    tt32 = token_type_ids.reshape(-1).astype(jnp.int32)
    return _run(ids32, tt32, tok_table, pos_table, ent_table,
                ln_gamma, ln_beta)


# speedup vs baseline: 9.1392x; 1.2351x over previous
import jax
import jax.numpy as jnp
from jax import lax
from jax.experimental import pallas as pl
from jax.experimental.pallas import tpu as pltpu
from jax.experimental.pallas import tpu_sc as plsc

B, S, V, P, D = 1024, 200, 50265, 514, 768
PAD_IDX = 1
N = B * S
NW = 32
NCHUNK = 4
BC = B // NCHUNK
NC_TOK = BC * S
KB = 80
PER_W = NC_TOK // NW
NBLK = PER_W // KB
BS_TC = 8


def _sc_gather_body(ids_hbm, tok_hbm, out_hbm,
                    idx0, idx1, buf0, buf1,
                    gsem0, gsem1, wsem0, wsem1):
    nc = 2
    wid = lax.axis_index("s") * nc + lax.axis_index("c")
    wbase = wid * PER_W

    idx = (idx0, idx1)
    buf = (buf0, buf1)
    gsem = (gsem0, gsem1)
    wsem = (wsem0, wsem1)

    pltpu.sync_copy(ids_hbm.at[pl.ds(wbase, KB)], idx0)
    pltpu.async_copy(tok_hbm.at[idx0], buf0, gsem0)

    def pair_body(h, _):
        for sub in (0, 1):
            g = 2 * h + sub
            cur, nxt = sub, 1 - sub

            def wait_prev_write():
                pltpu.make_async_copy(
                    buf[nxt], out_hbm.at[pl.ds(0, KB)], wsem[nxt]).wait()

            if sub == 1:
                wait_prev_write()
            else:
                pl.when(h > 0)(wait_prev_write)

            def launch_next():
                nbase = wbase + (g + 1) * KB
                pltpu.sync_copy(ids_hbm.at[pl.ds(nbase, KB)], idx[nxt])
                pltpu.async_copy(tok_hbm.at[idx[nxt]], buf[nxt], gsem[nxt])

            if sub == 0:
                launch_next()
            else:
                pl.when(h < NBLK // 2 - 1)(launch_next)

            pltpu.make_async_copy(
                tok_hbm.at[idx[cur]], buf[cur], gsem[cur]).wait()
            pltpu.async_copy(
                buf[cur], out_hbm.at[pl.ds(wbase + g * KB, KB)], wsem[cur])
        return 0

    lax.fori_loop(0, NBLK // 2, pair_body, 0)
    pltpu.make_async_copy(
        buf1, out_hbm.at[pl.ds(0, KB)], wsem1).wait()


def _tc_ln_body(g_ref, tt_ref, pos_ref, ent_ref, gam_ref, bet_ref, o_ref):
    x32 = g_ref[...]
    lo = lax.bitcast_convert_type(x32 << jnp.uint32(16), jnp.float32)
    hi = lax.bitcast_convert_type(x32 & jnp.uint32(0xFFFF0000), jnp.float32)
    x = jnp.concatenate([lo, hi], axis=-1) + pos_ref[...][None]
    ttf = tt_ref[...].astype(jnp.float32)[..., None]
    e0 = ent_ref[0, :][None, None, :]
    de = (ent_ref[1, :] - ent_ref[0, :])[None, None, :]
    x = x + e0 + ttf * de
    mean = jnp.mean(x, axis=-1, keepdims=True)
    xc = x - mean
    var = jnp.mean(xc * xc, axis=-1, keepdims=True)
    o_ref[...] = (xc * lax.rsqrt(var + 1e-5) * gam_ref[...][None, None, :]
                  + bet_ref[...][None, None, :])


def _tc_ln_body_aliased(g_ref, tt_ref, pos_ref, ent_ref, gam_ref, bet_ref,
                        buf_ref, o_ref):
    del buf_ref
    _tc_ln_body(g_ref, tt_ref, pos_ref, ent_ref, gam_ref, bet_ref, o_ref)


@jax.jit
def _run(ids32, tt32, tok_table, pos_table, ent_table, ln_gamma, ln_beta):
    mesh = plsc.VectorSubcoreMesh(core_axis_name="c", subcore_axis_name="s")
    gather = pl.kernel(
        _sc_gather_body,
        out_type=jax.ShapeDtypeStruct((NC_TOK, D // 2), jnp.uint32),
        mesh=mesh,
        scratch_types=[
            pltpu.VMEM((KB,), jnp.int32),
            pltpu.VMEM((KB,), jnp.int32),
            pltpu.VMEM((KB, D // 2), jnp.uint32),
            pltpu.VMEM((KB, D // 2), jnp.uint32),
            pltpu.SemaphoreType.DMA,
            pltpu.SemaphoreType.DMA,
            pltpu.SemaphoreType.DMA,
            pltpu.SemaphoreType.DMA,
        ],
    )

    lo16 = lax.bitcast_convert_type(
        tok_table[:, :D // 2].astype(jnp.bfloat16), jnp.uint16)
    hi16 = lax.bitcast_convert_type(
        tok_table[:, D // 2:].astype(jnp.bfloat16), jnp.uint16)
    tok_pk = (lo16.astype(jnp.uint32)
              | (hi16.astype(jnp.uint32) << jnp.uint32(16)))
    pos_slice = lax.slice_in_dim(pos_table, PAD_IDX + 1, PAD_IDX + 1 + S,
                                 axis=0)
    tt_bs = tt32.reshape(B, S)
    grid_c = BC // BS_TC

    def tc_chunk(c, rows_c, buf):
        common_in_specs = [
            pl.BlockSpec((BS_TC, S, D // 2), lambda i: (i, 0, 0)),
            pl.BlockSpec((BS_TC, S), lambda i, c=c: (c * grid_c + i, 0)),
            pl.BlockSpec((S, D), lambda i: (0, 0)),
            pl.BlockSpec((2, D), lambda i: (0, 0)),
            pl.BlockSpec((D,), lambda i: (0,)),
            pl.BlockSpec((D,), lambda i: (0,)),
        ]
        out_spec = pl.BlockSpec((BS_TC, S, D),
                                lambda i, c=c: (c * grid_c + i, 0, 0))
        args = (rows_c.reshape(BC, S, D // 2), tt_bs, pos_slice, ent_table,
                ln_gamma, ln_beta)
        if buf is None:
            return pl.pallas_call(
                _tc_ln_body,
                grid=(grid_c,),
                in_specs=common_in_specs,
                out_specs=out_spec,
                out_shape=jax.ShapeDtypeStruct((B, S, D), jnp.float32),
            )(*args)
        return pl.pallas_call(
            _tc_ln_body_aliased,
            grid=(grid_c,),
            in_specs=common_in_specs + [pl.BlockSpec(memory_space=pl.ANY)],
            out_specs=out_spec,
            out_shape=jax.ShapeDtypeStruct((B, S, D), jnp.float32),
            input_output_aliases={6: 0},
        )(*args, buf)

    out = None
    for c in range(NCHUNK):
        ids_c = lax.slice_in_dim(ids32, c * NC_TOK, (c + 1) * NC_TOK, axis=0)
        rows_c = gather(ids_c, tok_pk)
        out = tc_chunk(c, rows_c, out)
    return out


def kernel(input_ids, token_type_ids, tok_table, pos_table, ent_table,
           ln_gamma, ln_beta):
    ids32 = input_ids.reshape(-1).astype(jnp.int32)
    tt32 = token_type_ids.reshape(-1).astype(jnp.int32)
    return _run(ids32, tt32, tok_table, pos_table, ent_table,
                ln_gamma, ln_beta)
